# Initial kernel scaffold; baseline (speedup 1.0000x reference)
import jax
import jax.numpy as jnp
from jax.experimental import pallas as pl


def kernel(features, W, b):
    raise NotImplementedError("write your pallas kernel here")


# speedup vs baseline: 23.0302x; 23.0302x over previous
import jax
import jax.numpy as jnp
from jax import lax
from jax.experimental import pallas as pl

_K = 8
_R = 512


def _proj_kernel(x_ref, w_ref, b_ref, nodes_ref):
    x = x_ref[0]
    w = w_ref[...]
    bv = b_ref[...]
    nodes = lax.dot_general(
        x, w, (((0,), (1,)), ((), ())),
        preferred_element_type=jnp.float32)
    nodes_ref[0] = nodes + bv


def _topk_kernel(rows_ref, nodes_ref, idx_ref):
    rows = rows_ref[0]
    nodes = nodes_ref[0]
    xx_all = jnp.sum(nodes * nodes, axis=1)
    xx_rows = jnp.sum(rows * rows, axis=1)
    inner = lax.dot_general(
        rows, nodes, (((1,), (1,)), ((), ())),
        preferred_element_type=jnp.float32)
    d = (xx_rows[:, None] + (-2.0) * inner) + xx_all[None, :]
    n = d.shape[1]
    cols = lax.broadcasted_iota(jnp.int32, d.shape, 1)
    ams = []
    for t in range(_K + 1):
        m = jnp.min(d, axis=1)
        cand = jnp.where(d == m[:, None], cols, n)
        am = jnp.min(cand, axis=1)
        if t > 0:
            ams.append(am)
        if t < _K:
            d = jnp.where(cols == am[:, None], jnp.inf, d)
    idx_ref[0] = jnp.stack(ams, axis=1)


def kernel(features, W, b):
    B, C, H, Wd = features.shape
    N = H * Wd
    Cout = W.shape[0]
    x = features.reshape(B, C, N)
    Wm = W[:, :, 0, 0]
    b2 = b.reshape(1, Cout)

    nodes = pl.pallas_call(
        _proj_kernel,
        grid=(B,),
        in_specs=[pl.BlockSpec((1, C, N), lambda i: (i, 0, 0)),
                  pl.BlockSpec((Cout, C), lambda i: (0, 0)),
                  pl.BlockSpec((1, Cout), lambda i: (0, 0))],
        out_specs=pl.BlockSpec((1, N, Cout), lambda i: (i, 0, 0)),
        out_shape=jax.ShapeDtypeStruct((B, N, Cout), jnp.float32),
    )(x, Wm, b2)

    nb = N // _R
    idx = pl.pallas_call(
        _topk_kernel,
        grid=(B, nb),
        in_specs=[pl.BlockSpec((1, _R, Cout), lambda bi, ri: (bi, ri, 0)),
                  pl.BlockSpec((1, N, Cout), lambda bi, ri: (bi, 0, 0))],
        out_specs=pl.BlockSpec((1, _R, _K), lambda bi, ri: (bi, ri, 0)),
        out_shape=jax.ShapeDtypeStruct((B, N, _K), jnp.int32),
    )(nodes, nodes)

    src = jnp.broadcast_to(jnp.arange(N, dtype=idx.dtype)[None, :, None],
                           idx.shape)
    edge_index = jnp.stack([src, idx], axis=1).reshape(B, 2, N * _K)
    return (nodes, edge_index)


# speedup vs baseline: 27.1430x; 1.1786x over previous
import functools

import jax
import jax.numpy as jnp
from jax import lax
from jax.experimental import pallas as pl
from jax.experimental.pallas import tpu as pltpu
from jax.experimental.pallas import tpu_sc as plsc

_K = 8
_R = 1024


def _proj_kernel(x_ref, w_ref, b_ref, nodes_ref):
    x = x_ref[0]
    w = w_ref[...]
    bv = b_ref[...]
    nodes = lax.dot_general(
        x, w, (((0,), (1,)), ((), ())),
        preferred_element_type=jnp.float32)
    nodes_ref[0] = nodes + bv


def _topk_kernel(rows_ref, nodes_ref, idx_ref):
    rows = rows_ref[0]
    nodes = nodes_ref[0]
    xx_all = jnp.sum(nodes * nodes, axis=1)
    xx_rows = jnp.sum(rows * rows, axis=1)
    inner = lax.dot_general(
        rows, nodes, (((1,), (1,)), ((), ())),
        preferred_element_type=jnp.float32)
    d = (xx_rows[:, None] + (-2.0) * inner) + xx_all[None, :]
    n = float(d.shape[1])
    cols = lax.broadcasted_iota(jnp.int32, d.shape, 1).astype(jnp.float32)
    ams = []
    for t in range(_K + 1):
        m = jnp.min(d, axis=1)
        cand = jnp.where(d == m[:, None], cols, n)
        am = jnp.min(cand, axis=1)
        if t > 0:
            ams.append(am.astype(jnp.int32))
        if t < _K:
            d = jnp.where(cols == am[:, None], jnp.inf, d)
    idx_ref[0] = jnp.stack(ams, axis=1)


def _edge_assemble(idx_flat, batches, n, k):
    nk = n * k
    seg = nk // 8
    nv = seg // 16
    sh = k.bit_length() - 1
    mesh = plsc.VectorSubcoreMesh(core_axis_name="c", subcore_axis_name="s")

    @functools.partial(
        pl.kernel, mesh=mesh,
        out_type=jax.ShapeDtypeStruct((batches, 2, nk), jnp.int32),
        scratch_types=[pltpu.VMEM((seg,), jnp.int32),
                       pltpu.VMEM((seg,), jnp.int32)],
    )
    def k_fn(idx_hbm, out_hbm, dst_v, src_v):
        wid = lax.axis_index("s") * 2 + lax.axis_index("c")
        bi = wid // 8
        qi = wid % 8
        base = qi * seg
        lane = lax.broadcasted_iota(jnp.int32, (16,), 0)

        def body(j, _):
            slots = (base + j * 16) + lane
            src_v[pl.ds(j * 16, 16)] = lax.shift_right_logical(slots, sh)
            return _

        lax.fori_loop(0, nv, body, None)
        pltpu.sync_copy(idx_hbm.at[bi, pl.ds(base, seg)], dst_v)
        pltpu.sync_copy(src_v, out_hbm.at[bi, 0, pl.ds(base, seg)])
        pltpu.sync_copy(dst_v, out_hbm.at[bi, 1, pl.ds(base, seg)])

    return k_fn(idx_flat)


def kernel(features, W, b):
    B, C, H, Wd = features.shape
    N = H * Wd
    Cout = W.shape[0]
    x = features.reshape(B, C, N)
    Wm = W[:, :, 0, 0]
    b2 = b.reshape(1, Cout)

    nodes = pl.pallas_call(
        _proj_kernel,
        grid=(B,),
        in_specs=[pl.BlockSpec((1, C, N), lambda i: (i, 0, 0)),
                  pl.BlockSpec((Cout, C), lambda i: (0, 0)),
                  pl.BlockSpec((1, Cout), lambda i: (0, 0))],
        out_specs=pl.BlockSpec((1, N, Cout), lambda i: (i, 0, 0)),
        out_shape=jax.ShapeDtypeStruct((B, N, Cout), jnp.float32),
    )(x, Wm, b2)

    nb = N // _R
    idx = pl.pallas_call(
        _topk_kernel,
        grid=(B, nb),
        in_specs=[pl.BlockSpec((1, _R, Cout), lambda bi, ri: (bi, ri, 0)),
                  pl.BlockSpec((1, N, Cout), lambda bi, ri: (bi, 0, 0))],
        out_specs=pl.BlockSpec((1, _R, _K), lambda bi, ri: (bi, ri, 0)),
        out_shape=jax.ShapeDtypeStruct((B, N, _K), jnp.int32),
    )(nodes, nodes)

    edge_index = _edge_assemble(idx.reshape(B, N * _K), B, N, _K)
    return (nodes, edge_index)
